# baseline (speedup 1.0000x reference)
import functools
import math

import jax
import jax.numpy as jnp
from jax import lax
from jax.experimental import pallas as pl
from jax.experimental.pallas import tpu as pltpu
from jax.experimental.pallas import tpu_sc as plsc

LANES = 16

_info = plsc.get_sparse_core_info()
NUM_CORES = _info.num_cores
NUM_SUBCORES = _info.num_subcores
NUM_WORKERS = NUM_CORES * NUM_SUBCORES

BLK = 128


def _make_lookup(B, S, V, D):
    assert B % (BLK * NUM_WORKERS // NUM_WORKERS) == 0 and B % BLK == 0
    n_bblk = B // BLK
    assert n_bblk == NUM_WORKERS
    scale = math.sqrt(D)
    n_vecs_t = BLK // LANES
    mesh = plsc.VectorSubcoreMesh(core_axis_name="c", subcore_axis_name="s")

    @functools.partial(
        pl.kernel,
        mesh=mesh,
        compiler_params=pltpu.CompilerParams(
            use_tc_tiling_on_sc=True, needs_layout_passes=False
        ),
        out_type=jax.ShapeDtypeStruct((S, D, B), jnp.float32),
        scratch_types=[
            [pltpu.VMEM((BLK,), jnp.int32) for _ in range(2)],
            [pltpu.VMEM((BLK,), jnp.int32) for _ in range(2)],
            [pltpu.VMEM((BLK, 2 * D), jnp.float32) for _ in range(2)],
            [pltpu.VMEM((D, BLK), jnp.float32) for _ in range(2)],
            [pltpu.SemaphoreType.DMA for _ in range(2)],
            [pltpu.SemaphoreType.DMA for _ in range(2)],
        ],
    )
    def lookup(xt_hbm, lut2_hbm, out_hbm, idxs, hidxs, rows, slabs, gsem, ssem):
        wid = lax.axis_index("s") * NUM_CORES + lax.axis_index("c")
        bbase = wid * BLK

        def stage_idx(s, b):
            pltpu.sync_copy(xt_hbm.at[s, pl.ds(bbase, BLK)], idxs[b])
            for t in range(n_vecs_t):
                sl = pl.ds(t * LANES, LANES)
                hidxs[b][sl] = lax.shift_right_logical(idxs[b][sl], 1)

        def gather_copy(b):
            return pltpu.make_async_copy(lut2_hbm.at[hidxs[b]], rows[b], gsem[b])

        def slab_copy(s, b):
            return pltpu.make_async_copy(
                slabs[b], out_hbm.at[s, :, pl.ds(bbase, BLK)], ssem[b]
            )

        stage_idx(0, 0)
        gather_copy(0).start()

        def step(s, b, nb, first, last):
            @pl.when(jnp.logical_not(last))
            def _():
                stage_idx(s + 1, nb)
                @pl.when(jnp.logical_not(first))
                def _():
                    slab_copy(s - 1, nb).wait()

                gather_copy(nb).start()

            gather_copy(b).wait()

            sb = slabs[b]
            rb = rows[b]
            for t in range(n_vecs_t):
                sl = pl.ds(t * LANES, LANES)
                i_idx = lax.iota(jnp.int32, LANES) + t * LANES
                j_base = (idxs[b][sl] & 1) * D

                @plsc.parallel_loop(0, D, step=2, unroll=2)
                def _(d):
                    for dd in range(2):
                        v = plsc.load_gather(rb, [i_idx, j_base + (d + dd)])
                        sb[d + dd, sl] = v * scale

            slab_copy(s, b).start()

        def pair_body(g, carry):
            s0 = 2 * g
            step(s0, 0, 1, first=(g == 0), last=jnp.bool_(False))
            step(s0 + 1, 1, 0, first=jnp.bool_(False), last=(g == (S // 2 - 1)))
            return carry

        lax.fori_loop(0, S // 2, pair_body, 0)
        slab_copy(S - 2, 0).wait()
        slab_copy(S - 1, 1).wait()

    return lookup


def kernel(x, lut):
    B, S = x.shape
    V, D = lut.shape
    xt = x.T.astype(jnp.int32)
    lut2 = lut.reshape(V // 2, 2 * D)
    outp = _make_lookup(B, S, V, D)(xt, lut2)
    return outp.transpose(2, 0, 1)


# speedup vs baseline: 1.0531x; 1.0531x over previous
import functools
import math

import jax
import jax.numpy as jnp
from jax import lax
from jax.experimental import pallas as pl
from jax.experimental.pallas import tpu as pltpu
from jax.experimental.pallas import tpu_sc as plsc

LANES = 16

_info = plsc.get_sparse_core_info()
NUM_CORES = _info.num_cores
NUM_SUBCORES = _info.num_subcores
NUM_WORKERS = NUM_CORES * NUM_SUBCORES

BLK = 128
S_B = 2


def _make_lookup(B, S, V, D):
    assert B % BLK == 0 and B // BLK == NUM_WORKERS
    assert S % (2 * S_B) == 0
    n_batches = S // S_B
    scale = math.sqrt(D)
    n_vecs_t = BLK // LANES
    mesh = plsc.VectorSubcoreMesh(core_axis_name="c", subcore_axis_name="s")

    @functools.partial(
        pl.kernel,
        mesh=mesh,
        compiler_params=pltpu.CompilerParams(
            use_tc_tiling_on_sc=True, needs_layout_passes=False
        ),
        out_type=jax.ShapeDtypeStruct((S, D, B), jnp.float32),
        scratch_types=[
            [pltpu.VMEM((S_B, BLK), jnp.int32) for _ in range(2)],
            [pltpu.VMEM((S_B * BLK,), jnp.int32) for _ in range(2)],
            [pltpu.VMEM((S_B * BLK, 2 * D), jnp.float32) for _ in range(2)],
            [pltpu.VMEM((S_B, D, BLK), jnp.float32) for _ in range(2)],
            [pltpu.SemaphoreType.DMA for _ in range(2)],
            [pltpu.SemaphoreType.DMA for _ in range(2)],
        ],
    )
    def lookup(xt_hbm, lut2_hbm, out_hbm, idxs, hidxs, rows, slabs, gsem, ssem):
        wid = lax.axis_index("s") * NUM_CORES + lax.axis_index("c")
        bbase = wid * BLK

        def stage_idx(k, b):
            pltpu.sync_copy(
                xt_hbm.at[pl.ds(k * S_B, S_B), pl.ds(bbase, BLK)], idxs[b]
            )
            for sp in range(S_B):
                for t in range(n_vecs_t):
                    sl = pl.ds(t * LANES, LANES)
                    fl = pl.ds((sp * n_vecs_t + t) * LANES, LANES)
                    hidxs[b][fl] = lax.shift_right_logical(idxs[b][sp, sl], 1)

        def gather_copy(b):
            return pltpu.make_async_copy(lut2_hbm.at[hidxs[b]], rows[b], gsem[b])

        def slab_copy(k, b):
            return pltpu.make_async_copy(
                slabs[b],
                out_hbm.at[pl.ds(k * S_B, S_B), :, pl.ds(bbase, BLK)],
                ssem[b],
            )

        stage_idx(0, 0)
        gather_copy(0).start()

        def step(k, b, nb, first, last):
            @pl.when(jnp.logical_not(last))
            def _():
                stage_idx(k + 1, nb)

                @pl.when(jnp.logical_not(first))
                def _():
                    slab_copy(k - 1, nb).wait()

                gather_copy(nb).start()

            gather_copy(b).wait()

            rb = rows[b]
            sb = slabs[b]
            for sp in range(S_B):
                i_vecs = []
                j_vecs = []
                for t in range(n_vecs_t):
                    sl = pl.ds(t * LANES, LANES)
                    i_vecs.append(
                        lax.iota(jnp.int32, LANES) + (sp * n_vecs_t + t) * LANES
                    )
                    j_vecs.append((idxs[b][sp, sl] & 1) * D)

                @plsc.parallel_loop(0, D, step=2, unroll=1)
                def _(d):
                    for dd in range(2):
                        for t in range(n_vecs_t):
                            v = plsc.load_gather(rb, [i_vecs[t], j_vecs[t] + (d + dd)])
                            sb[sp, d + dd, pl.ds(t * LANES, LANES)] = v * scale

            slab_copy(k, b).start()

        def pair_body(g, carry):
            k0 = 2 * g
            step(k0, 0, 1, first=(g == 0), last=jnp.bool_(False))
            step(k0 + 1, 1, 0, first=jnp.bool_(False), last=(g == (n_batches // 2 - 1)))
            return carry

        lax.fori_loop(0, n_batches // 2, pair_body, 0)
        slab_copy(n_batches - 2, 0).wait()
        slab_copy(n_batches - 1, 1).wait()

    return lookup


def kernel(x, lut):
    B, S = x.shape
    V, D = lut.shape
    xt = x.T.astype(jnp.int32)
    lut2 = lut.reshape(V // 2, 2 * D)
    outp = _make_lookup(B, S, V, D)(xt, lut2)
    return outp.transpose(2, 0, 1)
